# baseline (speedup 1.0000x reference)
import jax
import jax.numpy as jnp
from jax.experimental import pallas as pl
from jax.experimental.pallas import tpu as pltpu


def _make_kernel(c, h, w):
    oh, ow = h // 2, w // 2
    npix = h * w

    def body(x_ref, we_ref, wo_ref, b_ref, o_ref, e0, e1, o0, o1):
        xb = x_ref[0].astype(jnp.bfloat16)
        zc = jnp.zeros((c, 2 * w), dtype=jnp.bfloat16)
        xp = jnp.concatenate([xb, zc], axis=1)
        sh2 = xp[:, 2 * w:]
        lhs_e = jnp.concatenate([xb, sh2], axis=0)

        ztail = jnp.zeros((2 * w, c // 2), dtype=jnp.float32)
        e0[npix:, :] = ztail
        e1[npix:, :] = ztail
        o0[npix:, :] = ztail
        o1[npix:, :] = ztail

        jidx = jax.lax.broadcasted_iota(jnp.int32, (oh, ow, c), 1)
        last_j = jidx == (ow - 1)

        acc = jnp.broadcast_to(b_ref[...], (oh * ow, c)).reshape(oh, ow, c)
        for kw in range(3):
            f = jax.lax.dot_general(
                lhs_e, we_ref[kw], (((0,), (0,)), ((), ())),
                preferred_element_type=jnp.float32)
            e0[:npix, :] = f[:, :c // 2]
            e1[:npix, :] = f[:, c // 2:]
            sel = jnp.concatenate(
                [e0[pl.ds(kw, npix // 2, 2), :],
                 e1[pl.ds(kw, npix // 2, 2), :]], axis=1)
            g = sel.reshape(oh, 2, ow, c)[:, 0]
            f1 = jax.lax.dot_general(
                xb, wo_ref[kw], (((0,), (0,)), ((), ())),
                preferred_element_type=jnp.float32)
            o0[:npix, :] = f1[:, :c // 2]
            o1[:npix, :] = f1[:, c // 2:]
            sel1 = jnp.concatenate(
                [o0[pl.ds(w + kw, npix // 2, 2), :],
                 o1[pl.ds(w + kw, npix // 2, 2), :]], axis=1)
            g = g + sel1.reshape(oh, 2, ow, c)[:, 0]
            if kw == 2:
                g = jnp.where(last_j, 0.0, g)
            acc = acc + g
        accT = jnp.transpose(acc.reshape(oh * ow, c), (1, 0))
        o_ref[...] = accT.reshape(1, c, oh, ow)

    return body


def kernel(x, weight, bias):
    n, c, h, w = x.shape
    oh, ow = h // 2, w // 2
    xf = x.reshape(n, c, h * w)

    w_hwio = jnp.transpose(weight, (2, 3, 1, 0)).astype(jnp.bfloat16)
    we = jnp.concatenate([w_hwio[0], w_hwio[2]], axis=1)
    wo = w_hwio[1]
    b2d = bias.reshape(1, c)

    flops = 2 * n * h * w * 9 * c * c // 2
    bytes_accessed = xf.size * 4 + n * oh * ow * c * 4

    out = pl.pallas_call(
        _make_kernel(c, h, w),
        out_shape=jax.ShapeDtypeStruct((n, c, oh, ow), jnp.float32),
        grid=(n,),
        in_specs=[
            pl.BlockSpec((1, c, h * w), lambda b: (b, 0, 0)),
            pl.BlockSpec((3, 2 * c, c), lambda b: (0, 0, 0)),
            pl.BlockSpec((3, c, c), lambda b: (0, 0, 0)),
            pl.BlockSpec((1, c), lambda b: (0, 0)),
        ],
        out_specs=pl.BlockSpec((1, c, oh, ow), lambda b: (b, 0, 0, 0)),
        scratch_shapes=[
            pltpu.VMEM((h * w + 2 * w, c // 2), jnp.float32),
            pltpu.VMEM((h * w + 2 * w, c // 2), jnp.float32),
            pltpu.VMEM((h * w + 2 * w, c // 2), jnp.float32),
            pltpu.VMEM((h * w + 2 * w, c // 2), jnp.float32),
        ],
        compiler_params=pltpu.CompilerParams(
            dimension_semantics=("parallel",),
            vmem_limit_bytes=48 * 1024 * 1024),
        cost_estimate=pl.CostEstimate(
            flops=flops, transcendentals=0, bytes_accessed=bytes_accessed),
    )(xf, we, wo, b2d)

    return out


# speedup vs baseline: 4.0994x; 4.0994x over previous
import jax
import jax.numpy as jnp
from jax.experimental import pallas as pl
from jax.experimental.pallas import tpu as pltpu


def _make_kernel(c, h, w):
    oh, ow = h // 2, w // 2
    m = oh * ow

    def body(x_ref, w2_ref, w1_ref, b_ref, o_ref):
        xb = x_ref[0].astype(jnp.bfloat16)
        xr = xb.reshape(oh, 2, w, c)
        rows0 = xr[:, 0]
        rows1 = xr[:, 1]
        rows2 = jnp.concatenate(
            [rows0[1:], jnp.zeros((1, w, c), dtype=jnp.bfloat16)], axis=0)

        zlane = jnp.zeros((oh, 2 * c), dtype=jnp.bfloat16)
        acc = jnp.broadcast_to(b_ref[...], (m, c))
        for kh, rows in enumerate((rows0, rows1, rows2)):
            v = rows.reshape(m, 2 * c)
            acc = acc + jnp.dot(v, w2_ref[kh],
                                preferred_element_type=jnp.float32)
            flat = rows.reshape(oh, w * c)
            u = jnp.concatenate([flat[:, 2 * c:], zlane], axis=1)
            z = u.reshape(m, 2 * c)[:, :c]
            acc = acc + jnp.dot(z, w1_ref[kh],
                                preferred_element_type=jnp.float32)
        o_ref[...] = acc.reshape(1, oh, ow, c)

    return body


def kernel(x, weight, bias):
    n, c, h, w = x.shape
    oh, ow = h // 2, w // 2

    xt = jnp.transpose(x, (0, 2, 3, 1))

    w_hwio = jnp.transpose(weight, (2, 3, 1, 0)).astype(jnp.bfloat16)
    w2 = jnp.concatenate([w_hwio[:, 0], w_hwio[:, 1]], axis=1)
    w1 = w_hwio[:, 2]
    b2d = bias.reshape(1, c)

    flops = 2 * n * oh * ow * 9 * c * c
    bytes_accessed = xt.size * 4 + n * oh * ow * c * 4

    out = pl.pallas_call(
        _make_kernel(c, h, w),
        out_shape=jax.ShapeDtypeStruct((n, oh, ow, c), jnp.float32),
        grid=(n,),
        in_specs=[
            pl.BlockSpec((1, h, w, c), lambda b: (b, 0, 0, 0)),
            pl.BlockSpec((3, 2 * c, c), lambda b: (0, 0, 0)),
            pl.BlockSpec((3, c, c), lambda b: (0, 0, 0)),
            pl.BlockSpec((1, c), lambda b: (0, 0)),
        ],
        out_specs=pl.BlockSpec((1, oh, ow, c), lambda b: (b, 0, 0, 0)),
        compiler_params=pltpu.CompilerParams(
            dimension_semantics=("parallel",),
            vmem_limit_bytes=48 * 1024 * 1024),
        cost_estimate=pl.CostEstimate(
            flops=flops, transcendentals=0, bytes_accessed=bytes_accessed),
    )(xt, w2, w1, b2d)

    return jnp.transpose(out, (0, 3, 1, 2))
